# baseline (speedup 1.0000x reference)
import functools

import jax
import jax.numpy as jnp
import numpy as np
from jax import lax
from jax.experimental import pallas as pl
from jax.experimental.pallas import tpu as pltpu
from jax.experimental.pallas import tpu_sc as plsc

N = 10000
NP = 10240
D = 128
E = 160000
EP = 163840
HS = (5, 2, 38)
OFF = (0, 5, 7, 45)
ENCS = ('op', 'pidx', 'pspell')

NC = 2
NS = 16
CH = 128
EPT = EP // NS
NCHUNK = EPT // CH
ROWS_PT = NP // NS
NNOTE = 5000
NOTE_CH = 128
NOTE_CHUNKS = (NNOTE + NOTE_CH - 1) // NOTE_CH
NB = 5

BR = 1024


def _pad2(v, r0, rtot, c0, ctot):
    rows, cols = v.shape
    return lax.pad(v, jnp.float32(0),
                   ((r0, rtot - r0 - rows, 0), (c0, ctot - c0 - cols, 0)))


def _tca_body(x_ref, wp_refs, bp_refs, wl_refs, wr_refs, t0_ref, t1_ref,
              xwr_ref, wp_s, bp_s, w2_s, wr_s):

    @pl.when(pl.program_id(0) == 0)
    def _():
        wp_s[...] = jnp.concatenate([r[...] for r in wp_refs], axis=1)
        bp_s[...] = jnp.concatenate([r[...] for r in bp_refs], axis=1)
        w2_s[...] = sum(
            _pad2(wl_refs[et * 3 + k][...], (et * 3 + k) * 128, 768,
                  et * 48 + OFF[k], 96)
            for et in range(2) for k in range(3))
        wr_s[...] = jnp.concatenate([r[...] for r in wr_refs], axis=1)

    x = x_ref[...]
    h = jnp.maximum(
        lax.dot_general(x, wp_s[...], (((1,), (0,)), ((), ())),
                        preferred_element_type=jnp.float32) + bp_s[...], 0.0)
    t = lax.dot_general(h, w2_s[...], (((1,), (0,)), ((), ())),
                        preferred_element_type=jnp.float32)
    ones45 = (lax.broadcasted_iota(jnp.int32, (BR, 48), 1) == 45).astype(jnp.float32)
    t0_ref[...] = t[:, :48] + ones45
    t1_ref[...] = t[:, 48:] + ones45
    xwr_ref[...] = lax.dot_general(x, wr_s[...], (((1,), (0,)), ((), ())),
                                   preferred_element_type=jnp.float32)


def _full(shape):
    return pl.BlockSpec(shape, lambda i: tuple(0 for _ in shape))


def _tca(x, wps, bps, wls, wrs):
    return pl.pallas_call(
        _tca_body,
        grid=(NP // BR,),
        in_specs=[
            pl.BlockSpec((BR, D), lambda i: (i, 0)),
            [_full((D, D)) for _ in wps],
            [_full((1, D)) for _ in bps],
            [_full(w.shape) for w in wls],
            [_full(w.shape) for w in wrs],
        ],
        out_specs=[
            pl.BlockSpec((BR, 48), lambda i: (i, 0)),
            pl.BlockSpec((BR, 48), lambda i: (i, 0)),
            pl.BlockSpec((BR, 90), lambda i: (i, 0)),
        ],
        out_shape=[
            jax.ShapeDtypeStruct((NP, 48), jnp.float32),
            jax.ShapeDtypeStruct((NP, 48), jnp.float32),
            jax.ShapeDtypeStruct((NP, 90), jnp.float32),
        ],
        scratch_shapes=[
            pltpu.VMEM((D, 768), jnp.float32),
            pltpu.VMEM((1, 768), jnp.float32),
            pltpu.VMEM((768, 96), jnp.float32),
            pltpu.VMEM((D, 90), jnp.float32),
        ],
    )(x, wps, bps, wls, wrs)


_SC_MESH = plsc.VectorSubcoreMesh(core_axis_name="c", subcore_axis_name="s")


@functools.partial(
    pl.kernel,
    out_type=jax.ShapeDtypeStruct((NC, NP, 48), jnp.float32),
    mesh=_SC_MESH,
    scratch_types=[
        pltpu.VMEM((NCHUNK, CH), jnp.int32),
        pltpu.VMEM((NCHUNK, CH), jnp.int32),
        [pltpu.VMEM((CH, 48), jnp.float32) for _ in range(NB)],
        pltpu.VMEM((NOTE_CHUNKS, NOTE_CH), jnp.int32),
        pltpu.VMEM((NOTE_CH, 48), jnp.float32),
        pltpu.VMEM_SHARED((NP, 48), jnp.float32),
        [pltpu.SemaphoreType.DMA for _ in range(NB)],
        [pltpu.SemaphoreType.DMA for _ in range(NB)],
    ],
    compiler_params=pltpu.CompilerParams(use_tc_tiling_on_sc=False),
)
def _sc_segsum(t0_hbm, t1_hbm, src0_hbm, dst0_hbm, src1_hbm, dst1_hbm,
               notes_hbm, zrows_hbm, mrows_hbm, out_hbm,
               src_v, dst_v, rows_bufs, nidx_v, mrow_v, acc, gsems, ssems):
    c = lax.axis_index("c")
    s = lax.axis_index("s")

    pltpu.sync_copy(zrows_hbm, acc.at[pl.ds(s * ROWS_PT, ROWS_PT)])
    plsc.subcore_barrier()

    def run(tab, src, dst):
        pltpu.sync_copy(src.at[s], src_v)
        pltpu.sync_copy(dst.at[s], dst_v)

        for b in range(NB):
            pltpu.async_copy(tab.at[src_v.at[b]], rows_bufs[b], gsems[b])

        def body(o, tok):
            for b in range(NB):
                i = o * NB + b
                pltpu.make_async_copy(tab.at[src_v.at[i]], rows_bufs[b],
                                      gsems[b]).wait()
                pltpu.sync_copy(rows_bufs[b], acc.at[dst_v.at[i]], add=True)
                nxt = i + NB

                @pl.when(nxt < NCHUNK)
                def _():
                    pltpu.async_copy(tab.at[src_v.at[nxt]], rows_bufs[b],
                                     gsems[b])
            return tok
        lax.fori_loop(0, NCHUNK // NB, body, 0)

    @pl.when(c == 0)
    def _():
        run(t0_hbm, src0_hbm, dst0_hbm)

    @pl.when(c == 1)
    def _():
        run(t1_hbm, src1_hbm, dst1_hbm)

    w = c * NS + s
    pltpu.sync_copy(mrows_hbm, mrow_v)
    pltpu.sync_copy(notes_hbm, nidx_v)

    def note_chunk(ci):
        pltpu.sync_copy(mrow_v, acc.at[nidx_v.at[ci]], add=True)

    note_chunk(w)

    @pl.when(w + NC * NS < NOTE_CHUNKS)
    def _():
        note_chunk(w + NC * NS)

    plsc.subcore_barrier()
    pltpu.sync_copy(acc.at[pl.ds(s * ROWS_PT, ROWS_PT)],
                    out_hbm.at[c].at[pl.ds(s * ROWS_PT, ROWS_PT)])


def _tcb_body(s0_ref, xwr_ref, bl_refs, g_refs, b_refs, smat_ref, smatt_ref,
              winv_ref, out_ref):
    smat = smat_ref[...]
    smatt = smatt_ref[...]

    def seg_bcast(v3):
        return lax.dot_general(v3, smatt, (((1,), (0,)), ((), ())),
                               preferred_element_type=jnp.float32)

    def seg_sum(v45):
        return lax.dot_general(v45, smat, (((1,), (0,)), ((), ())),
                               preferred_element_type=jnp.float32)

    def half(sx, xwr_half, bl_row):
        cnt = jnp.maximum(sx[:, 45:46], 1.0)
        o = sx[:, :45] / cnt + xwr_half + bl_row
        den = seg_bcast(jnp.maximum(jnp.sqrt(seg_sum(o * o)), 1e-12))
        return o / den

    xwr = xwr_ref[...]
    bl0 = jnp.concatenate([r[...] for r in bl_refs[0:3]], axis=1)
    bl1 = jnp.concatenate([r[...] for r in bl_refs[3:6]], axis=1)
    o0 = half(s0_ref[0], xwr[:, :45], bl0)
    o1 = half(s0_ref[1], xwr[:, 45:], bl1)
    h = jnp.maximum((o0 + o1) * 0.5, 0.0)
    winv = winv_ref[...]
    mu = seg_sum(h) * winv
    ex2 = seg_sum(h * h) * winv
    var = ex2 - mu * mu
    g = jnp.concatenate([r[...] for r in g_refs], axis=1)
    b = jnp.concatenate([r[...] for r in b_refs], axis=1)
    ln = (h - seg_bcast(mu)) * jax.lax.rsqrt(seg_bcast(var) + 1e-5) * g + b
    pad = (lax.broadcasted_iota(jnp.int32, (BR, 3), 1) == 0).astype(jnp.float32)
    out_ref[...] = jnp.concatenate([ln, pad], axis=1)


def _tcb(s0, xwr, bls, gs, bs, smat, smatt, winv):
    return pl.pallas_call(
        _tcb_body,
        grid=(NP // BR,),
        in_specs=[
            pl.BlockSpec((NC, BR, 48), lambda i: (0, i, 0)),
            pl.BlockSpec((BR, 90), lambda i: (i, 0)),
            [_full(a.shape) for a in bls],
            [_full(a.shape) for a in gs],
            [_full(a.shape) for a in bs],
            _full((45, 3)),
            _full((3, 45)),
            _full((1, 3)),
        ],
        out_specs=pl.BlockSpec((BR, 48), lambda i: (i, 0)),
        out_shape=jax.ShapeDtypeStruct((NP, 48), jnp.float32),
    )(s0, xwr, bls, gs, bs, smat, smatt, winv)


def _tcc_body(s1_ref, s0_ref, h1_ref, wl1_refs, wr1_refs, bl1_refs, out_ref,
              bdl0_s, bdl1_s, bdr_s):
    @pl.when(pl.program_id(0) == 0)
    def _():
        bdl0_s[...] = sum(_pad2(wl1_refs[k][...], OFF[k], 45, OFF[k], 45)
                          for k in range(3))
        bdl1_s[...] = sum(_pad2(wl1_refs[3 + k][...], OFF[k], 45, OFF[k], 45)
                          for k in range(3))
        bdr_s[...] = sum(
            _pad2(0.5 * (wr1_refs[k][...] + wr1_refs[3 + k][...]),
                  OFF[k], 45, OFF[k], 45) for k in range(3))

    s0a, s0b = s0_ref[0], s0_ref[1]
    cnt0 = jnp.maximum(s0a[:, 45:46], 1.0)
    cnt1 = jnp.maximum(s0b[:, 45:46], 1.0)
    m0 = s1_ref[0][:, :45] / cnt0
    m1 = s1_ref[1][:, :45] / cnt1

    def mm(a, b_ref):
        return lax.dot_general(a, b_ref[...], (((1,), (0,)), ((), ())),
                               preferred_element_type=jnp.float32)

    bl1 = 0.5 * (jnp.concatenate([r[...] for r in bl1_refs[0:3]], axis=1)
                 + jnp.concatenate([r[...] for r in bl1_refs[3:6]], axis=1))
    out = (0.5 * (mm(m0, bdl0_s) + mm(m1, bdl1_s))
           + mm(h1_ref[:, :45], bdr_s) + bl1)
    mask = (s0a[:, 46:47] + s0b[:, 46:47]) > 0.0
    out_ref[...] = jnp.where(mask, out, 0.0)


def _tcc(s1, s0, h1t, wl1s, wr1s, bl1s):
    return pl.pallas_call(
        _tcc_body,
        grid=(NP // BR,),
        in_specs=[
            pl.BlockSpec((NC, BR, 48), lambda i: (0, i, 0)),
            pl.BlockSpec((NC, BR, 48), lambda i: (0, i, 0)),
            pl.BlockSpec((BR, 48), lambda i: (i, 0)),
            [_full(a.shape) for a in wl1s],
            [_full(a.shape) for a in wr1s],
            [_full(a.shape) for a in bl1s],
        ],
        out_specs=pl.BlockSpec((BR, 45), lambda i: (i, 0)),
        out_shape=jax.ShapeDtypeStruct((NP, 45), jnp.float32),
        scratch_shapes=[
            pltpu.VMEM((45, 45), jnp.float32),
            pltpu.VMEM((45, 45), jnp.float32),
            pltpu.VMEM((45, 45), jnp.float32),
        ],
    )(s1, s0, h1t, wl1s, wr1s, bl1s)


_SMAT = np.zeros((45, 3), np.float32)
for _k in range(3):
    _SMAT[OFF[_k]:OFF[_k + 1], _k] = 1.0
_WINV = (1.0 / np.array(HS, np.float32))[None, :]
_MROWS = np.zeros((NOTE_CH, 48), np.float32)
_MROWS[:, 46] = 1.0


def kernel(x, edge_index_onset, edge_index_consecutive, ts_beats, divs_pq,
           onset_div, duration_div, not_removed_notes, computation_notes,
           target, params):
    l0 = [params[e]['l0'][et] for et in range(2) for e in ENCS]
    l1 = [params[e]['l1'][et] for et in range(2) for e in ENCS]
    wps = [p['Wp'] for p in l0]
    bps = [p['bp'][None, :] for p in l0]
    wls = [p['Wl'] for p in l0]
    wrs = [p['Wr'] for p in l0]
    bls = [p['bl'][None, :] for p in l0]
    gs = [params[e]['ln0_g'][None, :] for e in ENCS]
    bs = [params[e]['ln0_b'][None, :] for e in ENCS]
    wl1s = [p['Wl'] for p in l1]
    wr1s = [p['Wr'] for p in l1]
    bl1s = [p['bl'][None, :] for p in l1]
    smat = jnp.asarray(_SMAT)
    smatt = jnp.asarray(np.ascontiguousarray(_SMAT.T))
    winv = jnp.asarray(_WINV)
    mrows = jnp.asarray(_MROWS)
    zrows = jnp.zeros((ROWS_PT, 48), jnp.float32)

    eip0 = jnp.pad(edge_index_onset, ((0, 0), (0, EP - E)),
                   constant_values=NP - 1)
    eip1 = jnp.pad(edge_index_consecutive, ((0, 0), (0, EP - E)),
                   constant_values=NP - 1)
    src0 = eip0[0].reshape(NS, NCHUNK, CH)
    dst0 = eip0[1].reshape(NS, NCHUNK, CH)
    src1 = eip1[0].reshape(NS, NCHUNK, CH)
    dst1 = eip1[1].reshape(NS, NCHUNK, CH)
    notes = jnp.pad(computation_notes.astype(jnp.int32),
                    (0, NOTE_CH * NOTE_CHUNKS - NNOTE),
                    mode='edge').reshape(NOTE_CHUNKS, NOTE_CH)

    t0, t1, xwr = _tca(x, wps, bps, wls, wrs)
    s0 = _sc_segsum(t0, t1, src0, dst0, src1, dst1, notes, zrows, mrows)
    h1t = _tcb(s0, xwr, bls, gs, bs, smat, smatt, winv)
    s1 = _sc_segsum(h1t, h1t, src0, dst0, src1, dst1, notes, zrows, mrows)
    return _tcc(s1, s0, h1t, wl1s, wr1s, bl1s)[:N]


# speedup vs baseline: 1.7774x; 1.7774x over previous
import functools

import jax
import jax.numpy as jnp
import numpy as np
from jax import lax
from jax.experimental import pallas as pl
from jax.experimental.pallas import tpu as pltpu
from jax.experimental.pallas import tpu_sc as plsc

N = 10000
NP = 10240
D = 128
E = 160000
EP = 163840
HS = (5, 2, 38)
OFF = (0, 5, 7, 45)
ENCS = ('op', 'pidx', 'pspell')

NC = 2
NS = 16
CH = 128
EPT = EP // NS
NCHUNK = EPT // CH
ROWS_PT = NP // NS
NNOTE = 5000
NOTE_CH = 128
NOTE_CHUNKS = (NNOTE + NOTE_CH - 1) // NOTE_CH
NB = 5

BR = 1024


def _pad2(v, r0, rtot, c0, ctot):
    rows, cols = v.shape
    return lax.pad(v, jnp.float32(0),
                   ((r0, rtot - r0 - rows, 0), (c0, ctot - c0 - cols, 0)))


def _tca_body(x_ref, wp_refs, bp_refs, wl_refs, wr_refs, t0_ref, t1_ref,
              xwr_ref, wp_s, bp_s, w2_s, wr_s):

    @pl.when(pl.program_id(0) == 0)
    def _():
        wp_s[...] = jnp.concatenate([r[...] for r in wp_refs], axis=1)
        bp_s[...] = jnp.concatenate([r[...] for r in bp_refs], axis=1)
        w2_s[...] = sum(
            _pad2(wl_refs[et * 3 + k][...], (et * 3 + k) * 128, 768,
                  et * 48 + OFF[k], 96)
            for et in range(2) for k in range(3))
        wr_s[...] = jnp.concatenate([r[...] for r in wr_refs], axis=1)

    x = x_ref[...]
    h = jnp.maximum(
        lax.dot_general(x, wp_s[...], (((1,), (0,)), ((), ())),
                        preferred_element_type=jnp.float32) + bp_s[...], 0.0)
    t = lax.dot_general(h, w2_s[...], (((1,), (0,)), ((), ())),
                        preferred_element_type=jnp.float32)
    ones45 = (lax.broadcasted_iota(jnp.int32, (BR, 48), 1) == 45).astype(jnp.float32)
    t0_ref[...] = t[:, :48] + ones45
    t1_ref[...] = t[:, 48:] + ones45
    xwr_ref[...] = lax.dot_general(x, wr_s[...], (((1,), (0,)), ((), ())),
                                   preferred_element_type=jnp.float32)


def _full(shape):
    return pl.BlockSpec(shape, lambda i: tuple(0 for _ in shape))


def _tca(x, wps, bps, wls, wrs):
    return pl.pallas_call(
        _tca_body,
        grid=(NP // BR,),
        in_specs=[
            pl.BlockSpec((BR, D), lambda i: (i, 0)),
            [_full((D, D)) for _ in wps],
            [_full((1, D)) for _ in bps],
            [_full(w.shape) for w in wls],
            [_full(w.shape) for w in wrs],
        ],
        out_specs=[
            pl.BlockSpec((BR, 48), lambda i: (i, 0)),
            pl.BlockSpec((BR, 48), lambda i: (i, 0)),
            pl.BlockSpec((BR, 90), lambda i: (i, 0)),
        ],
        out_shape=[
            jax.ShapeDtypeStruct((NP, 48), jnp.float32),
            jax.ShapeDtypeStruct((NP, 48), jnp.float32),
            jax.ShapeDtypeStruct((NP, 90), jnp.float32),
        ],
        scratch_shapes=[
            pltpu.VMEM((D, 768), jnp.float32),
            pltpu.VMEM((1, 768), jnp.float32),
            pltpu.VMEM((768, 96), jnp.float32),
            pltpu.VMEM((D, 90), jnp.float32),
        ],
    )(x, wps, bps, wls, wrs)


_SC_MESH = plsc.VectorSubcoreMesh(core_axis_name="c", subcore_axis_name="s")


@functools.partial(
    pl.kernel,
    out_type=jax.ShapeDtypeStruct((NC, NP, 48), jnp.float32),
    mesh=_SC_MESH,
    scratch_types=[
        pltpu.VMEM((NCHUNK, CH), jnp.int32),
        pltpu.VMEM((NCHUNK, CH), jnp.int32),
        [pltpu.VMEM((CH, 48), jnp.float32) for _ in range(NB)],
        pltpu.VMEM((NOTE_CHUNKS, NOTE_CH), jnp.int32),
        pltpu.VMEM((NOTE_CH, 48), jnp.float32),
        pltpu.VMEM_SHARED((NP, 48), jnp.float32),
        [pltpu.SemaphoreType.DMA for _ in range(NB)],
        [pltpu.SemaphoreType.DMA for _ in range(NB)],
    ],
    compiler_params=pltpu.CompilerParams(use_tc_tiling_on_sc=False),
)
def _sc_segsum(t0_hbm, t1_hbm, src0_hbm, dst0_hbm, src1_hbm, dst1_hbm,
               notes_hbm, zrows_hbm, mrows_hbm, out_hbm,
               src_v, dst_v, rows_bufs, nidx_v, mrow_v, acc, gsems, ssems):
    c = lax.axis_index("c")
    s = lax.axis_index("s")

    pltpu.sync_copy(zrows_hbm, acc.at[pl.ds(s * ROWS_PT, ROWS_PT)])
    plsc.subcore_barrier()

    def run(tab, src, dst):
        pltpu.sync_copy(src.at[s], src_v)
        pltpu.sync_copy(dst.at[s], dst_v)

        for b in range(NB):
            pltpu.async_copy(tab.at[src_v.at[b]], rows_bufs[b], gsems[b])

        def body(o, tok):
            for b in range(NB):
                i = o * NB + b
                pltpu.make_async_copy(tab.at[src_v.at[i]], rows_bufs[b],
                                      gsems[b]).wait()
                pltpu.sync_copy(rows_bufs[b], acc.at[dst_v.at[i]], add=True)
                nxt = i + NB

                @pl.when(nxt < NCHUNK)
                def _():
                    pltpu.async_copy(tab.at[src_v.at[nxt]], rows_bufs[b],
                                     gsems[b])
            return tok
        lax.fori_loop(0, NCHUNK // NB, body, 0)

    @pl.when(c == 0)
    def _():
        run(t0_hbm, src0_hbm, dst0_hbm)

    @pl.when(c == 1)
    def _():
        run(t1_hbm, src1_hbm, dst1_hbm)

    w = c * NS + s
    pltpu.sync_copy(mrows_hbm, mrow_v)
    pltpu.sync_copy(notes_hbm, nidx_v)

    def note_chunk(ci):
        pltpu.sync_copy(mrow_v, acc.at[nidx_v.at[ci]], add=True)

    note_chunk(w)

    @pl.when(w + NC * NS < NOTE_CHUNKS)
    def _():
        note_chunk(w + NC * NS)

    plsc.subcore_barrier()
    pltpu.sync_copy(acc.at[pl.ds(s * ROWS_PT, ROWS_PT)],
                    out_hbm.at[c].at[pl.ds(s * ROWS_PT, ROWS_PT)])


def _tcb_body(s0_ref, xwr_ref, bl_refs, g_refs, b_refs, smat_ref, smatt_ref,
              winv_ref, out_ref):
    smat = smat_ref[...]
    smatt = smatt_ref[...]

    def seg_bcast(v3):
        return lax.dot_general(v3, smatt, (((1,), (0,)), ((), ())),
                               preferred_element_type=jnp.float32)

    def seg_sum(v45):
        return lax.dot_general(v45, smat, (((1,), (0,)), ((), ())),
                               preferred_element_type=jnp.float32)

    def half(sx, xwr_half, bl_row):
        cnt = jnp.maximum(sx[:, 45:46], 1.0)
        o = sx[:, :45] / cnt + xwr_half + bl_row
        den = seg_bcast(jnp.maximum(jnp.sqrt(seg_sum(o * o)), 1e-12))
        return o / den

    xwr = xwr_ref[...]
    bl0 = jnp.concatenate([r[...] for r in bl_refs[0:3]], axis=1)
    bl1 = jnp.concatenate([r[...] for r in bl_refs[3:6]], axis=1)
    o0 = half(s0_ref[0], xwr[:, :45], bl0)
    o1 = half(s0_ref[1], xwr[:, 45:], bl1)
    h = jnp.maximum((o0 + o1) * 0.5, 0.0)
    winv = winv_ref[...]
    mu = seg_sum(h) * winv
    ex2 = seg_sum(h * h) * winv
    var = ex2 - mu * mu
    g = jnp.concatenate([r[...] for r in g_refs], axis=1)
    b = jnp.concatenate([r[...] for r in b_refs], axis=1)
    ln = (h - seg_bcast(mu)) * jax.lax.rsqrt(seg_bcast(var) + 1e-5) * g + b
    pad = (lax.broadcasted_iota(jnp.int32, (BR, 3), 1) == 0).astype(jnp.float32)
    out_ref[...] = jnp.concatenate([ln, pad], axis=1)


def _tcb(s0, xwr, bls, gs, bs, smat, smatt, winv):
    return pl.pallas_call(
        _tcb_body,
        grid=(NP // BR,),
        in_specs=[
            pl.BlockSpec((NC, BR, 48), lambda i: (0, i, 0)),
            pl.BlockSpec((BR, 90), lambda i: (i, 0)),
            [_full(a.shape) for a in bls],
            [_full(a.shape) for a in gs],
            [_full(a.shape) for a in bs],
            _full((45, 3)),
            _full((3, 45)),
            _full((1, 3)),
        ],
        out_specs=pl.BlockSpec((BR, 48), lambda i: (i, 0)),
        out_shape=jax.ShapeDtypeStruct((NP, 48), jnp.float32),
    )(s0, xwr, bls, gs, bs, smat, smatt, winv)


def _tcc_body(s1_ref, s0_ref, h1_ref, wl1_refs, wr1_refs, bl1_refs, out_ref,
              bdl0_s, bdl1_s, bdr_s):
    @pl.when(pl.program_id(0) == 0)
    def _():
        bdl0_s[...] = sum(_pad2(wl1_refs[k][...], OFF[k], 45, OFF[k], 45)
                          for k in range(3))
        bdl1_s[...] = sum(_pad2(wl1_refs[3 + k][...], OFF[k], 45, OFF[k], 45)
                          for k in range(3))
        bdr_s[...] = sum(
            _pad2(0.5 * (wr1_refs[k][...] + wr1_refs[3 + k][...]),
                  OFF[k], 45, OFF[k], 45) for k in range(3))

    s0a, s0b = s0_ref[0], s0_ref[1]
    cnt0 = jnp.maximum(s0a[:, 45:46], 1.0)
    cnt1 = jnp.maximum(s0b[:, 45:46], 1.0)
    m0 = s1_ref[0][:, :45] / cnt0
    m1 = s1_ref[1][:, :45] / cnt1

    def mm(a, b_ref):
        return lax.dot_general(a, b_ref[...], (((1,), (0,)), ((), ())),
                               preferred_element_type=jnp.float32)

    bl1 = 0.5 * (jnp.concatenate([r[...] for r in bl1_refs[0:3]], axis=1)
                 + jnp.concatenate([r[...] for r in bl1_refs[3:6]], axis=1))
    out = (0.5 * (mm(m0, bdl0_s) + mm(m1, bdl1_s))
           + mm(h1_ref[:, :45], bdr_s) + bl1)
    mask = (s0a[:, 46:47] + s0b[:, 46:47]) > 0.0
    out_ref[...] = jnp.where(mask, out, 0.0)


def _tcc(s1, s0, h1t, wl1s, wr1s, bl1s):
    return pl.pallas_call(
        _tcc_body,
        grid=(NP // BR,),
        in_specs=[
            pl.BlockSpec((NC, BR, 48), lambda i: (0, i, 0)),
            pl.BlockSpec((NC, BR, 48), lambda i: (0, i, 0)),
            pl.BlockSpec((BR, 48), lambda i: (i, 0)),
            [_full(a.shape) for a in wl1s],
            [_full(a.shape) for a in wr1s],
            [_full(a.shape) for a in bl1s],
        ],
        out_specs=pl.BlockSpec((BR, 45), lambda i: (i, 0)),
        out_shape=jax.ShapeDtypeStruct((NP, 45), jnp.float32),
        scratch_shapes=[
            pltpu.VMEM((45, 45), jnp.float32),
            pltpu.VMEM((45, 45), jnp.float32),
            pltpu.VMEM((45, 45), jnp.float32),
        ],
    )(s1, s0, h1t, wl1s, wr1s, bl1s)


_SMAT = np.zeros((45, 3), np.float32)
for _k in range(3):
    _SMAT[OFF[_k]:OFF[_k + 1], _k] = 1.0
_WINV = (1.0 / np.array(HS, np.float32))[None, :]
_MROWS = np.zeros((NOTE_CH, 48), np.float32)
_MROWS[:, 46] = 1.0
_EPAD = np.broadcast_to(N + np.arange(EP - E, dtype=np.int32) % (NP - N),
                        (2, EP - E))


def kernel(x, edge_index_onset, edge_index_consecutive, ts_beats, divs_pq,
           onset_div, duration_div, not_removed_notes, computation_notes,
           target, params):
    l0 = [params[e]['l0'][et] for et in range(2) for e in ENCS]
    l1 = [params[e]['l1'][et] for et in range(2) for e in ENCS]
    wps = [p['Wp'] for p in l0]
    bps = [p['bp'][None, :] for p in l0]
    wls = [p['Wl'] for p in l0]
    wrs = [p['Wr'] for p in l0]
    bls = [p['bl'][None, :] for p in l0]
    gs = [params[e]['ln0_g'][None, :] for e in ENCS]
    bs = [params[e]['ln0_b'][None, :] for e in ENCS]
    wl1s = [p['Wl'] for p in l1]
    wr1s = [p['Wr'] for p in l1]
    bl1s = [p['bl'][None, :] for p in l1]
    smat = jnp.asarray(_SMAT)
    smatt = jnp.asarray(np.ascontiguousarray(_SMAT.T))
    winv = jnp.asarray(_WINV)
    mrows = jnp.asarray(_MROWS)
    zrows = jnp.zeros((ROWS_PT, 48), jnp.float32)

    epad = jnp.asarray(_EPAD)
    eip0 = jnp.concatenate([edge_index_onset, epad], axis=1)
    eip1 = jnp.concatenate([edge_index_consecutive, epad], axis=1)
    src0 = eip0[0].reshape(NS, NCHUNK, CH)
    dst0 = eip0[1].reshape(NS, NCHUNK, CH)
    src1 = eip1[0].reshape(NS, NCHUNK, CH)
    dst1 = eip1[1].reshape(NS, NCHUNK, CH)
    notes = jnp.pad(computation_notes.astype(jnp.int32),
                    (0, NOTE_CH * NOTE_CHUNKS - NNOTE),
                    mode='edge').reshape(NOTE_CHUNKS, NOTE_CH)

    t0, t1, xwr = _tca(x, wps, bps, wls, wrs)
    s0 = _sc_segsum(t0, t1, src0, dst0, src1, dst1, notes, zrows, mrows)
    h1t = _tcb(s0, xwr, bls, gs, bs, smat, smatt, winv)
    s1 = _sc_segsum(h1t, h1t, src0, dst0, src1, dst1, notes, zrows, mrows)
    return _tcc(s1, s0, h1t, wl1s, wr1s, bl1s)[:N]


# speedup vs baseline: 1.8348x; 1.0323x over previous
import functools

import jax
import jax.numpy as jnp
import numpy as np
from jax import lax
from jax.experimental import pallas as pl
from jax.experimental.pallas import tpu as pltpu
from jax.experimental.pallas import tpu_sc as plsc

N = 10000
NP = 10240
D = 128
E = 160000
EP = 163840
HS = (5, 2, 38)
OFF = (0, 5, 7, 45)
ENCS = ('op', 'pidx', 'pspell')

NC = 2
NS = 16
CH = 128
EPT = EP // NS
NCHUNK = EPT // CH
ROWS_PT = NP // NS
NNOTE = 5000
NOTE_CH = 128
NOTE_CHUNKS = (NNOTE + NOTE_CH - 1) // NOTE_CH
NB = 8

BR = 1024


def _pad2(v, r0, rtot, c0, ctot):
    rows, cols = v.shape
    return lax.pad(v, jnp.float32(0),
                   ((r0, rtot - r0 - rows, 0), (c0, ctot - c0 - cols, 0)))


_WLOFF = (0, 5, 7, 45, 50, 52, 90)


def _tca_body(x_ref, wp_refs, bp_refs, wlr_ref, t0_ref, t1_ref,
              xwr_ref, wp_s, bp_s, w2_s, wr_s):

    @pl.when(pl.program_id(0) == 0)
    def _():
        wp_s[...] = jnp.concatenate([r[...] for r in wp_refs], axis=1)
        bp_s[...] = jnp.concatenate([r[...] for r in bp_refs], axis=1)
        w2_s[...] = sum(
            _pad2(wlr_ref[:, _WLOFF[et * 3 + k]:_WLOFF[et * 3 + k + 1]],
                  (et * 3 + k) * 128, 768, et * 48 + OFF[k], 96)
            for et in range(2) for k in range(3))
        wr_s[...] = wlr_ref[:, 90:180]

    x = x_ref[...]
    h = jnp.maximum(
        lax.dot_general(x, wp_s[...], (((1,), (0,)), ((), ())),
                        preferred_element_type=jnp.float32) + bp_s[...], 0.0)
    t = lax.dot_general(h, w2_s[...], (((1,), (0,)), ((), ())),
                        preferred_element_type=jnp.float32)
    ones45 = (lax.broadcasted_iota(jnp.int32, (BR, 48), 1) == 45).astype(jnp.float32)
    t0_ref[...] = t[:, :48] + ones45
    t1_ref[...] = t[:, 48:] + ones45
    xwr_ref[...] = lax.dot_general(x, wr_s[...], (((1,), (0,)), ((), ())),
                                   preferred_element_type=jnp.float32)


def _full(shape):
    return pl.BlockSpec(shape, lambda i: tuple(0 for _ in shape))


def _tca(x, wps, bps, wlr):
    return pl.pallas_call(
        _tca_body,
        grid=(NP // BR,),
        in_specs=[
            pl.BlockSpec((BR, D), lambda i: (i, 0)),
            [_full((D, D)) for _ in wps],
            [_full((1, D)) for _ in bps],
            _full((D, 180)),
        ],
        out_specs=[
            pl.BlockSpec((BR, 48), lambda i: (i, 0)),
            pl.BlockSpec((BR, 48), lambda i: (i, 0)),
            pl.BlockSpec((BR, 90), lambda i: (i, 0)),
        ],
        out_shape=[
            jax.ShapeDtypeStruct((NP, 48), jnp.float32),
            jax.ShapeDtypeStruct((NP, 48), jnp.float32),
            jax.ShapeDtypeStruct((NP, 90), jnp.float32),
        ],
        scratch_shapes=[
            pltpu.VMEM((D, 768), jnp.float32),
            pltpu.VMEM((1, 768), jnp.float32),
            pltpu.VMEM((768, 96), jnp.float32),
            pltpu.VMEM((D, 90), jnp.float32),
        ],
    )(x, wps, bps, wlr)


_SC_MESH = plsc.VectorSubcoreMesh(core_axis_name="c", subcore_axis_name="s")


@functools.partial(
    pl.kernel,
    out_type=jax.ShapeDtypeStruct((NC, NP, 48), jnp.float32),
    mesh=_SC_MESH,
    scratch_types=[
        pltpu.VMEM((NCHUNK, CH), jnp.int32),
        pltpu.VMEM((NCHUNK, CH), jnp.int32),
        [pltpu.VMEM((CH, 48), jnp.float32) for _ in range(NB)],
        pltpu.VMEM((NOTE_CHUNKS, NOTE_CH), jnp.int32),
        pltpu.VMEM((NOTE_CH, 48), jnp.float32),
        pltpu.VMEM_SHARED((NP, 48), jnp.float32),
        [pltpu.SemaphoreType.DMA for _ in range(NB)],
        [pltpu.SemaphoreType.DMA for _ in range(NB)],
    ],
    compiler_params=pltpu.CompilerParams(use_tc_tiling_on_sc=False),
)
def _sc_segsum(t0_hbm, t1_hbm, src0_hbm, dst0_hbm, src1_hbm, dst1_hbm,
               notes_hbm, zrows_hbm, mrows_hbm, out_hbm,
               src_v, dst_v, rows_bufs, nidx_v, mrow_v, acc, gsems, ssems):
    c = lax.axis_index("c")
    s = lax.axis_index("s")

    pltpu.sync_copy(zrows_hbm, acc.at[pl.ds(s * ROWS_PT, ROWS_PT)])
    plsc.subcore_barrier()

    def run(tab, src, dst):
        pltpu.sync_copy(src.at[s], src_v)
        pltpu.sync_copy(dst.at[s], dst_v)

        for b in range(NB):
            pltpu.async_copy(tab.at[src_v.at[b]], rows_bufs[b], gsems[b])

        def body(o, tok):
            for b in range(NB):
                i = o * NB + b
                pltpu.make_async_copy(tab.at[src_v.at[i]], rows_bufs[b],
                                      gsems[b]).wait()
                pltpu.sync_copy(rows_bufs[b], acc.at[dst_v.at[i]], add=True)
                nxt = i + NB

                @pl.when(nxt < NCHUNK)
                def _():
                    pltpu.async_copy(tab.at[src_v.at[nxt]], rows_bufs[b],
                                     gsems[b])
            return tok
        lax.fori_loop(0, NCHUNK // NB, body, 0)

    @pl.when(c == 0)
    def _():
        run(t0_hbm, src0_hbm, dst0_hbm)

    @pl.when(c == 1)
    def _():
        run(t1_hbm, src1_hbm, dst1_hbm)

    w = c * NS + s
    pltpu.sync_copy(mrows_hbm, mrow_v)
    pltpu.sync_copy(notes_hbm, nidx_v)

    def note_chunk(ci):
        pltpu.sync_copy(mrow_v, acc.at[nidx_v.at[ci]], add=True)

    note_chunk(w)

    @pl.when(w + NC * NS < NOTE_CHUNKS)
    def _():
        note_chunk(w + NC * NS)

    plsc.subcore_barrier()
    pltpu.sync_copy(acc.at[pl.ds(s * ROWS_PT, ROWS_PT)],
                    out_hbm.at[c].at[pl.ds(s * ROWS_PT, ROWS_PT)])


def _tcb_body(s0_ref, xwr_ref, bl_refs, g_refs, b_refs, smat_ref, smatt_ref,
              winv_ref, out_ref):
    smat = smat_ref[...]
    smatt = smatt_ref[...]

    def seg_bcast(v3):
        return lax.dot_general(v3, smatt, (((1,), (0,)), ((), ())),
                               preferred_element_type=jnp.float32)

    def seg_sum(v45):
        return lax.dot_general(v45, smat, (((1,), (0,)), ((), ())),
                               preferred_element_type=jnp.float32)

    def half(sx, xwr_half, bl_row):
        cnt = jnp.maximum(sx[:, 45:46], 1.0)
        o = sx[:, :45] / cnt + xwr_half + bl_row
        den = seg_bcast(jnp.maximum(jnp.sqrt(seg_sum(o * o)), 1e-12))
        return o / den

    xwr = xwr_ref[...]
    bl0 = jnp.concatenate([r[...] for r in bl_refs[0:3]], axis=1)
    bl1 = jnp.concatenate([r[...] for r in bl_refs[3:6]], axis=1)
    o0 = half(s0_ref[0], xwr[:, :45], bl0)
    o1 = half(s0_ref[1], xwr[:, 45:], bl1)
    h = jnp.maximum((o0 + o1) * 0.5, 0.0)
    winv = winv_ref[...]
    mu = seg_sum(h) * winv
    ex2 = seg_sum(h * h) * winv
    var = ex2 - mu * mu
    g = jnp.concatenate([r[...] for r in g_refs], axis=1)
    b = jnp.concatenate([r[...] for r in b_refs], axis=1)
    ln = (h - seg_bcast(mu)) * jax.lax.rsqrt(seg_bcast(var) + 1e-5) * g + b
    pad = (lax.broadcasted_iota(jnp.int32, (BR, 3), 1) == 0).astype(jnp.float32)
    out_ref[...] = jnp.concatenate([ln, pad], axis=1)


def _tcb(s0, xwr, bls, gs, bs, smat, smatt, winv):
    return pl.pallas_call(
        _tcb_body,
        grid=(NP // BR,),
        in_specs=[
            pl.BlockSpec((NC, BR, 48), lambda i: (0, i, 0)),
            pl.BlockSpec((BR, 90), lambda i: (i, 0)),
            [_full(a.shape) for a in bls],
            [_full(a.shape) for a in gs],
            [_full(a.shape) for a in bs],
            _full((45, 3)),
            _full((3, 45)),
            _full((1, 3)),
        ],
        out_specs=pl.BlockSpec((BR, 48), lambda i: (i, 0)),
        out_shape=jax.ShapeDtypeStruct((NP, 48), jnp.float32),
    )(s0, xwr, bls, gs, bs, smat, smatt, winv)


def _tcc_body(s1_ref, s0_ref, h1_ref, wl1_refs, wr1_refs, bl1_refs, out_ref,
              bdl0_s, bdl1_s, bdr_s):
    @pl.when(pl.program_id(0) == 0)
    def _():
        bdl0_s[...] = sum(_pad2(wl1_refs[k][...], OFF[k], 45, OFF[k], 45)
                          for k in range(3))
        bdl1_s[...] = sum(_pad2(wl1_refs[3 + k][...], OFF[k], 45, OFF[k], 45)
                          for k in range(3))
        bdr_s[...] = sum(
            _pad2(0.5 * (wr1_refs[k][...] + wr1_refs[3 + k][...]),
                  OFF[k], 45, OFF[k], 45) for k in range(3))

    s0a, s0b = s0_ref[0], s0_ref[1]
    cnt0 = jnp.maximum(s0a[:, 45:46], 1.0)
    cnt1 = jnp.maximum(s0b[:, 45:46], 1.0)
    m0 = s1_ref[0][:, :45] / cnt0
    m1 = s1_ref[1][:, :45] / cnt1

    def mm(a, b_ref):
        return lax.dot_general(a, b_ref[...], (((1,), (0,)), ((), ())),
                               preferred_element_type=jnp.float32)

    bl1 = 0.5 * (jnp.concatenate([r[...] for r in bl1_refs[0:3]], axis=1)
                 + jnp.concatenate([r[...] for r in bl1_refs[3:6]], axis=1))
    out = (0.5 * (mm(m0, bdl0_s) + mm(m1, bdl1_s))
           + mm(h1_ref[:, :45], bdr_s) + bl1)
    mask = (s0a[:, 46:47] + s0b[:, 46:47]) > 0.0
    out_ref[...] = jnp.where(mask, out, 0.0)


def _tcc(s1, s0, h1t, wl1s, wr1s, bl1s):
    return pl.pallas_call(
        _tcc_body,
        grid=(NP // BR,),
        in_specs=[
            pl.BlockSpec((NC, BR, 48), lambda i: (0, i, 0)),
            pl.BlockSpec((NC, BR, 48), lambda i: (0, i, 0)),
            pl.BlockSpec((BR, 48), lambda i: (i, 0)),
            [_full(a.shape) for a in wl1s],
            [_full(a.shape) for a in wr1s],
            [_full(a.shape) for a in bl1s],
        ],
        out_specs=pl.BlockSpec((BR, 45), lambda i: (i, 0)),
        out_shape=jax.ShapeDtypeStruct((NP, 45), jnp.float32),
        scratch_shapes=[
            pltpu.VMEM((45, 45), jnp.float32),
            pltpu.VMEM((45, 45), jnp.float32),
            pltpu.VMEM((45, 45), jnp.float32),
        ],
    )(s1, s0, h1t, wl1s, wr1s, bl1s)


_SMAT = np.zeros((45, 3), np.float32)
for _k in range(3):
    _SMAT[OFF[_k]:OFF[_k + 1], _k] = 1.0
_WINV = (1.0 / np.array(HS, np.float32))[None, :]
_MROWS = np.zeros((NOTE_CH, 48), np.float32)
_MROWS[:, 46] = 1.0
_EPAD = np.broadcast_to(N + np.arange(EP - E, dtype=np.int32) % (NP - N),
                        (2, EP - E))


def kernel(x, edge_index_onset, edge_index_consecutive, ts_beats, divs_pq,
           onset_div, duration_div, not_removed_notes, computation_notes,
           target, params):
    l0 = [params[e]['l0'][et] for et in range(2) for e in ENCS]
    l1 = [params[e]['l1'][et] for et in range(2) for e in ENCS]
    wps = [p['Wp'] for p in l0]
    bps = [p['bp'][None, :] for p in l0]
    wls = [p['Wl'] for p in l0]
    wrs = [p['Wr'] for p in l0]
    bls = [p['bl'][None, :] for p in l0]
    gs = [params[e]['ln0_g'][None, :] for e in ENCS]
    bs = [params[e]['ln0_b'][None, :] for e in ENCS]
    wl1s = [p['Wl'] for p in l1]
    wr1s = [p['Wr'] for p in l1]
    bl1s = [p['bl'][None, :] for p in l1]
    smat = jnp.asarray(_SMAT)
    smatt = jnp.asarray(np.ascontiguousarray(_SMAT.T))
    winv = jnp.asarray(_WINV)
    mrows = jnp.asarray(_MROWS)
    zrows = jnp.zeros((ROWS_PT, 48), jnp.float32)

    epad = jnp.asarray(_EPAD)
    eip0 = jnp.concatenate([edge_index_onset, epad], axis=1)
    eip1 = jnp.concatenate([edge_index_consecutive, epad], axis=1)
    src0 = eip0[0].reshape(NS, NCHUNK, CH)
    dst0 = eip0[1].reshape(NS, NCHUNK, CH)
    src1 = eip1[0].reshape(NS, NCHUNK, CH)
    dst1 = eip1[1].reshape(NS, NCHUNK, CH)
    notes = jnp.pad(computation_notes.astype(jnp.int32),
                    (0, NOTE_CH * NOTE_CHUNKS - NNOTE),
                    mode='edge').reshape(NOTE_CHUNKS, NOTE_CH)

    wlr = jnp.concatenate(wls + wrs, axis=1)
    t0, t1, xwr = _tca(x, wps, bps, wlr)
    s0 = _sc_segsum(t0, t1, src0, dst0, src1, dst1, notes, zrows, mrows)
    h1t = _tcb(s0, xwr, bls, gs, bs, smat, smatt, winv)
    s1 = _sc_segsum(h1t, h1t, src0, dst0, src1, dst1, notes, zrows, mrows)
    return _tcc(s1, s0, h1t, wl1s, wr1s, bl1s)[:N]


# speedup vs baseline: 1.8918x; 1.0310x over previous
import functools

import jax
import jax.numpy as jnp
import numpy as np
from jax import lax
from jax.experimental import pallas as pl
from jax.experimental.pallas import tpu as pltpu
from jax.experimental.pallas import tpu_sc as plsc

N = 10000
NP = 10240
D = 128
E = 160000
EP = 163840
HS = (5, 2, 38)
OFF = (0, 5, 7, 45)
ENCS = ('op', 'pidx', 'pspell')

NC = 2
NS = 16
CH = 128
EPT = EP // NS
NCHUNK = EPT // CH
ROWS_PT = NP // NS
NNOTE = 5000
NOTE_CH = 128
NOTE_CHUNKS = (NNOTE + NOTE_CH - 1) // NOTE_CH
NB = 8

BR = 2048


def _pad2(v, r0, rtot, c0, ctot):
    rows, cols = v.shape
    return lax.pad(v, jnp.float32(0),
                   ((r0, rtot - r0 - rows, 0), (c0, ctot - c0 - cols, 0)))


_WLOFF = (0, 5, 7, 45, 50, 52, 90)


def _tca_body(x_ref, wp_refs, bp_refs, wlr_ref, t0_ref, t1_ref,
              xwr_ref, wp_s, bp_s, w2_s, wr_s):

    @pl.when(pl.program_id(0) == 0)
    def _():
        wp_s[...] = jnp.concatenate([r[...] for r in wp_refs], axis=1)
        bp_s[...] = jnp.concatenate([r[...] for r in bp_refs], axis=1)
        w2_s[...] = sum(
            _pad2(wlr_ref[:, _WLOFF[et * 3 + k]:_WLOFF[et * 3 + k + 1]],
                  (et * 3 + k) * 128, 768, et * 48 + OFF[k], 96)
            for et in range(2) for k in range(3))
        wr_s[...] = wlr_ref[:, 90:180]

    x = x_ref[...]
    h = jnp.maximum(
        lax.dot_general(x, wp_s[...], (((1,), (0,)), ((), ())),
                        preferred_element_type=jnp.float32) + bp_s[...], 0.0)
    t = lax.dot_general(h, w2_s[...], (((1,), (0,)), ((), ())),
                        preferred_element_type=jnp.float32)
    ones45 = (lax.broadcasted_iota(jnp.int32, (BR, 48), 1) == 45).astype(jnp.float32)
    t0_ref[...] = t[:, :48] + ones45
    t1_ref[...] = t[:, 48:] + ones45
    xwr_ref[...] = lax.dot_general(x, wr_s[...], (((1,), (0,)), ((), ())),
                                   preferred_element_type=jnp.float32)


def _full(shape):
    return pl.BlockSpec(shape, lambda i: tuple(0 for _ in shape))


def _tca(x, wps, bps, wlr):
    return pl.pallas_call(
        _tca_body,
        grid=(NP // BR,),
        in_specs=[
            pl.BlockSpec((BR, D), lambda i: (i, 0)),
            [_full((D, D)) for _ in wps],
            [_full((1, D)) for _ in bps],
            _full((D, 180)),
        ],
        out_specs=[
            pl.BlockSpec((BR, 48), lambda i: (i, 0)),
            pl.BlockSpec((BR, 48), lambda i: (i, 0)),
            pl.BlockSpec((BR, 90), lambda i: (i, 0)),
        ],
        out_shape=[
            jax.ShapeDtypeStruct((NP, 48), jnp.float32),
            jax.ShapeDtypeStruct((NP, 48), jnp.float32),
            jax.ShapeDtypeStruct((NP, 90), jnp.float32),
        ],
        scratch_shapes=[
            pltpu.VMEM((D, 768), jnp.float32),
            pltpu.VMEM((1, 768), jnp.float32),
            pltpu.VMEM((768, 96), jnp.float32),
            pltpu.VMEM((D, 90), jnp.float32),
        ],
    )(x, wps, bps, wlr)


_SC_MESH = plsc.VectorSubcoreMesh(core_axis_name="c", subcore_axis_name="s")


@functools.partial(
    pl.kernel,
    out_type=jax.ShapeDtypeStruct((NC, NP, 48), jnp.float32),
    mesh=_SC_MESH,
    scratch_types=[
        pltpu.VMEM((NCHUNK, CH), jnp.int32),
        pltpu.VMEM((NCHUNK, CH), jnp.int32),
        [pltpu.VMEM((CH, 48), jnp.float32) for _ in range(NB)],
        pltpu.VMEM((NOTE_CHUNKS, NOTE_CH), jnp.int32),
        pltpu.VMEM((NOTE_CH, 48), jnp.float32),
        pltpu.VMEM_SHARED((NP, 48), jnp.float32),
        [pltpu.SemaphoreType.DMA for _ in range(NB)],
        [pltpu.SemaphoreType.DMA for _ in range(NB)],
    ],
    compiler_params=pltpu.CompilerParams(use_tc_tiling_on_sc=False),
)
def _sc_segsum(t0_hbm, t1_hbm, src0_hbm, dst0_hbm, src1_hbm, dst1_hbm,
               notes_hbm, zrows_hbm, mrows_hbm, out_hbm,
               src_v, dst_v, rows_bufs, nidx_v, mrow_v, acc, gsems, ssems):
    c = lax.axis_index("c")
    s = lax.axis_index("s")

    pltpu.sync_copy(zrows_hbm, acc.at[pl.ds(s * ROWS_PT, ROWS_PT)])
    plsc.subcore_barrier()

    def run(tab, src, dst):
        pltpu.sync_copy(src.at[s], src_v)
        pltpu.sync_copy(dst.at[s], dst_v)

        for b in range(NB):
            pltpu.async_copy(tab.at[src_v.at[b]], rows_bufs[b], gsems[b])

        def body(o, tok):
            for b in range(NB):
                i = o * NB + b
                pltpu.make_async_copy(tab.at[src_v.at[i]], rows_bufs[b],
                                      gsems[b]).wait()
                pltpu.sync_copy(rows_bufs[b], acc.at[dst_v.at[i]], add=True)
                nxt = i + NB

                @pl.when(nxt < NCHUNK)
                def _():
                    pltpu.async_copy(tab.at[src_v.at[nxt]], rows_bufs[b],
                                     gsems[b])
            return tok
        lax.fori_loop(0, NCHUNK // NB, body, 0)

    @pl.when(c == 0)
    def _():
        run(t0_hbm, src0_hbm, dst0_hbm)

    @pl.when(c == 1)
    def _():
        run(t1_hbm, src1_hbm, dst1_hbm)

    w = c * NS + s
    pltpu.sync_copy(mrows_hbm, mrow_v)
    pltpu.sync_copy(notes_hbm, nidx_v)

    def note_chunk(ci):
        pltpu.sync_copy(mrow_v, acc.at[nidx_v.at[ci]], add=True)

    note_chunk(w)

    @pl.when(w + NC * NS < NOTE_CHUNKS)
    def _():
        note_chunk(w + NC * NS)

    plsc.subcore_barrier()
    pltpu.sync_copy(acc.at[pl.ds(s * ROWS_PT, ROWS_PT)],
                    out_hbm.at[c].at[pl.ds(s * ROWS_PT, ROWS_PT)])


def _tcb_body(s0_ref, xwr_ref, bl_refs, g_refs, b_refs, smat_ref, smatt_ref,
              winv_ref, out_ref):
    smat = smat_ref[...]
    smatt = smatt_ref[...]

    def seg_bcast(v3):
        return lax.dot_general(v3, smatt, (((1,), (0,)), ((), ())),
                               preferred_element_type=jnp.float32)

    def seg_sum(v45):
        return lax.dot_general(v45, smat, (((1,), (0,)), ((), ())),
                               preferred_element_type=jnp.float32)

    def half(sx, xwr_half, bl_row):
        cnt = jnp.maximum(sx[:, 45:46], 1.0)
        o = sx[:, :45] / cnt + xwr_half + bl_row
        den = seg_bcast(jnp.maximum(jnp.sqrt(seg_sum(o * o)), 1e-12))
        return o / den

    xwr = xwr_ref[...]
    bl0 = jnp.concatenate([r[...] for r in bl_refs[0:3]], axis=1)
    bl1 = jnp.concatenate([r[...] for r in bl_refs[3:6]], axis=1)
    o0 = half(s0_ref[0], xwr[:, :45], bl0)
    o1 = half(s0_ref[1], xwr[:, 45:], bl1)
    h = jnp.maximum((o0 + o1) * 0.5, 0.0)
    winv = winv_ref[...]
    mu = seg_sum(h) * winv
    ex2 = seg_sum(h * h) * winv
    var = ex2 - mu * mu
    g = jnp.concatenate([r[...] for r in g_refs], axis=1)
    b = jnp.concatenate([r[...] for r in b_refs], axis=1)
    ln = (h - seg_bcast(mu)) * jax.lax.rsqrt(seg_bcast(var) + 1e-5) * g + b
    pad = (lax.broadcasted_iota(jnp.int32, (BR, 3), 1) == 0).astype(jnp.float32)
    out_ref[...] = jnp.concatenate([ln, pad], axis=1)


def _tcb(s0, xwr, bls, gs, bs, smat, smatt, winv):
    return pl.pallas_call(
        _tcb_body,
        grid=(NP // BR,),
        in_specs=[
            pl.BlockSpec((NC, BR, 48), lambda i: (0, i, 0)),
            pl.BlockSpec((BR, 90), lambda i: (i, 0)),
            [_full(a.shape) for a in bls],
            [_full(a.shape) for a in gs],
            [_full(a.shape) for a in bs],
            _full((45, 3)),
            _full((3, 45)),
            _full((1, 3)),
        ],
        out_specs=pl.BlockSpec((BR, 48), lambda i: (i, 0)),
        out_shape=jax.ShapeDtypeStruct((NP, 48), jnp.float32),
    )(s0, xwr, bls, gs, bs, smat, smatt, winv)


def _tcc_body(s1_ref, s0_ref, h1_ref, wl1_refs, wr1_refs, bl1_refs, out_ref,
              bdl0_s, bdl1_s, bdr_s):
    @pl.when(pl.program_id(0) == 0)
    def _():
        bdl0_s[...] = sum(_pad2(wl1_refs[k][...], OFF[k], 45, OFF[k], 45)
                          for k in range(3))
        bdl1_s[...] = sum(_pad2(wl1_refs[3 + k][...], OFF[k], 45, OFF[k], 45)
                          for k in range(3))
        bdr_s[...] = sum(
            _pad2(0.5 * (wr1_refs[k][...] + wr1_refs[3 + k][...]),
                  OFF[k], 45, OFF[k], 45) for k in range(3))

    s0a, s0b = s0_ref[0], s0_ref[1]
    cnt0 = jnp.maximum(s0a[:, 45:46], 1.0)
    cnt1 = jnp.maximum(s0b[:, 45:46], 1.0)
    m0 = s1_ref[0][:, :45] / cnt0
    m1 = s1_ref[1][:, :45] / cnt1

    def mm(a, b_ref):
        return lax.dot_general(a, b_ref[...], (((1,), (0,)), ((), ())),
                               preferred_element_type=jnp.float32)

    bl1 = 0.5 * (jnp.concatenate([r[...] for r in bl1_refs[0:3]], axis=1)
                 + jnp.concatenate([r[...] for r in bl1_refs[3:6]], axis=1))
    out = (0.5 * (mm(m0, bdl0_s) + mm(m1, bdl1_s))
           + mm(h1_ref[:, :45], bdr_s) + bl1)
    mask = (s0a[:, 46:47] + s0b[:, 46:47]) > 0.0
    out_ref[...] = jnp.where(mask, out, 0.0)


def _tcc(s1, s0, h1t, wl1s, wr1s, bl1s):
    return pl.pallas_call(
        _tcc_body,
        grid=(NP // BR,),
        in_specs=[
            pl.BlockSpec((NC, BR, 48), lambda i: (0, i, 0)),
            pl.BlockSpec((NC, BR, 48), lambda i: (0, i, 0)),
            pl.BlockSpec((BR, 48), lambda i: (i, 0)),
            [_full(a.shape) for a in wl1s],
            [_full(a.shape) for a in wr1s],
            [_full(a.shape) for a in bl1s],
        ],
        out_specs=pl.BlockSpec((BR, 45), lambda i: (i, 0)),
        out_shape=jax.ShapeDtypeStruct((N, 45), jnp.float32),
        scratch_shapes=[
            pltpu.VMEM((45, 45), jnp.float32),
            pltpu.VMEM((45, 45), jnp.float32),
            pltpu.VMEM((45, 45), jnp.float32),
        ],
    )(s1, s0, h1t, wl1s, wr1s, bl1s)


_SMAT = np.zeros((45, 3), np.float32)
for _k in range(3):
    _SMAT[OFF[_k]:OFF[_k + 1], _k] = 1.0
_WINV = (1.0 / np.array(HS, np.float32))[None, :]
_MROWS = np.zeros((NOTE_CH, 48), np.float32)
_MROWS[:, 46] = 1.0
_EPAD = np.broadcast_to(N + np.arange(EP - E, dtype=np.int32) % (NP - N),
                        (2, EP - E))


def kernel(x, edge_index_onset, edge_index_consecutive, ts_beats, divs_pq,
           onset_div, duration_div, not_removed_notes, computation_notes,
           target, params):
    l0 = [params[e]['l0'][et] for et in range(2) for e in ENCS]
    l1 = [params[e]['l1'][et] for et in range(2) for e in ENCS]
    wps = [p['Wp'] for p in l0]
    bps = [p['bp'][None, :] for p in l0]
    wls = [p['Wl'] for p in l0]
    wrs = [p['Wr'] for p in l0]
    bls = [p['bl'][None, :] for p in l0]
    gs = [params[e]['ln0_g'][None, :] for e in ENCS]
    bs = [params[e]['ln0_b'][None, :] for e in ENCS]
    wl1s = [p['Wl'] for p in l1]
    wr1s = [p['Wr'] for p in l1]
    bl1s = [p['bl'][None, :] for p in l1]
    smat = jnp.asarray(_SMAT)
    smatt = jnp.asarray(np.ascontiguousarray(_SMAT.T))
    winv = jnp.asarray(_WINV)
    mrows = jnp.asarray(_MROWS)
    zrows = jnp.zeros((ROWS_PT, 48), jnp.float32)

    epad = jnp.asarray(_EPAD)
    eip0 = jnp.concatenate([edge_index_onset, epad], axis=1)
    eip1 = jnp.concatenate([edge_index_consecutive, epad], axis=1)
    src0 = eip0[0].reshape(NS, NCHUNK, CH)
    dst0 = eip0[1].reshape(NS, NCHUNK, CH)
    src1 = eip1[0].reshape(NS, NCHUNK, CH)
    dst1 = eip1[1].reshape(NS, NCHUNK, CH)
    notes = jnp.pad(computation_notes.astype(jnp.int32),
                    (0, NOTE_CH * NOTE_CHUNKS - NNOTE),
                    mode='edge').reshape(NOTE_CHUNKS, NOTE_CH)

    wlr = jnp.concatenate(wls + wrs, axis=1)
    t0, t1, xwr = _tca(x, wps, bps, wlr)
    s0 = _sc_segsum(t0, t1, src0, dst0, src1, dst1, notes, zrows, mrows)
    h1t = _tcb(s0, xwr, bls, gs, bs, smat, smatt, winv)
    s1 = _sc_segsum(h1t, h1t, src0, dst0, src1, dst1, notes, zrows, mrows)
    return _tcc(s1, s0, h1t, wl1s, wr1s, bl1s)
